# baseline (speedup 1.0000x reference)
import functools

import jax
import jax.numpy as jnp
from jax import lax
from jax.experimental import pallas as pl
from jax.experimental.pallas import tpu as pltpu
from jax.experimental.pallas import tpu_sc as plsc

_N = 10000
_E = 320000
_D = 128
_H = 16
_G = 64
_CLS = 2

_NC = 2
_NS = 16
_L = 16

_N_ACC = 10112
_ACC_PER_SUB = _N_ACC // _NS
_E_CHUNK = 128
_CHUNKS_PER_TILE = 80
_E_TILE = _CHUNKS_PER_TILE * _E_CHUNK
_E_PAD = _NC * _NS * _E_TILE

_STG_BIG = 632
_STG_LAST = _N - _STG_BIG * (_NS - 1)

_BR = 2000
_TC_GRID = _N // _BR

_NBUF = 4


def _sc_mesh():
    return plsc.VectorSubcoreMesh(core_axis_name="c", subcore_axis_name="s")


def _acc_writeback(acc, out0, out1, c, s):
    sl = pl.ds(s * _ACC_PER_SUB, _ACC_PER_SUB)

    @pl.when(c == 0)
    def _w0():
        pltpu.sync_copy(acc.at[sl], out0.at[sl])

    @pl.when(c == 1)
    def _w1():
        pltpu.sync_copy(acc.at[sl], out1.at[sl])


@functools.cache
def _make_deg_kernel():
    @functools.partial(
        pl.kernel,
        out_type=(
            jax.ShapeDtypeStruct((_N_ACC, _L), jnp.float32),
            jax.ShapeDtypeStruct((_N_ACC, _L), jnp.float32),
        ),
        mesh=_sc_mesh(),
        scratch_types=[
            pltpu.VMEM((_E_TILE,), jnp.int32),
            pltpu.VMEM((_E_CHUNK, _L), jnp.float32),
            pltpu.VMEM_SHARED((_N_ACC, _L), jnp.float32),
        ],
        compiler_params=pltpu.CompilerParams(use_tc_tiling_on_sc=False),
    )
    def deg_kernel(dst_hbm, ones_hbm, zeros_hbm, out0, out1, dst_v, ones_v, acc):
        c = lax.axis_index("c")
        s = lax.axis_index("s")
        gtile = c * _NS + s
        pltpu.sync_copy(dst_hbm.at[pl.ds(gtile * _E_TILE, _E_TILE)], dst_v)
        pltpu.sync_copy(ones_hbm, ones_v)
        pltpu.sync_copy(zeros_hbm, acc.at[pl.ds(s * _ACC_PER_SUB, _ACC_PER_SUB)])
        plsc.subcore_barrier()

        def body(j, carry):
            pltpu.sync_copy(ones_v, acc.at[dst_v.at[pl.ds(j * _E_CHUNK, _E_CHUNK)]],
                            add=True)
            return carry

        lax.fori_loop(0, _CHUNKS_PER_TILE, body, 0)
        plsc.subcore_barrier()
        _acc_writeback(acc, out0, out1, c, s)

    return deg_kernel


def _deg_kernel(dst_flat, ones, zeros):
    return _make_deg_kernel()(dst_flat, ones, zeros)


@functools.cache
def _make_edge_kernel():
    @functools.partial(
        pl.kernel,
        out_type=(
            jax.ShapeDtypeStruct((_N_ACC, _L), jnp.float32),
            jax.ShapeDtypeStruct((_N_ACC, _L), jnp.float32),
        ),
        mesh=_sc_mesh(),
        scratch_types=[
            pltpu.VMEM((_E_TILE,), jnp.int32),
            pltpu.VMEM((_E_TILE,), jnp.int32),
            pltpu.VMEM((_NBUF, _E_CHUNK, _L), jnp.float32),
        ]
        + [pltpu.SemaphoreType.DMA] * _NBUF
        + [
            pltpu.VMEM_SHARED((_N_ACC, _L), jnp.float32),
            pltpu.VMEM_SHARED((_N, _L), jnp.float32),
        ],
        compiler_params=pltpu.CompilerParams(use_tc_tiling_on_sc=False),
    )
    def edge_kernel(src_hbm, dst_hbm, rows_hbm, zeros_hbm, out0, out1,
                    src_v, dst_v, rows_v, sem0, sem1, sem2, sem3, acc, hn_s):
        sems = (sem0, sem1, sem2, sem3)
        c = lax.axis_index("c")
        s = lax.axis_index("s")
        gtile = c * _NS + s
        pltpu.sync_copy(src_hbm.at[pl.ds(gtile * _E_TILE, _E_TILE)], src_v)
        pltpu.sync_copy(dst_hbm.at[pl.ds(gtile * _E_TILE, _E_TILE)], dst_v)
        pltpu.sync_copy(zeros_hbm, acc.at[pl.ds(s * _ACC_PER_SUB, _ACC_PER_SUB)])

        @pl.when(s < _NS - 1)
        def _stage_big():
            sl = pl.ds(s * _STG_BIG, _STG_BIG)
            pltpu.sync_copy(rows_hbm.at[sl], hn_s.at[sl])

        @pl.when(s == _NS - 1)
        def _stage_last():
            sl = pl.ds((_NS - 1) * _STG_BIG, _STG_LAST)
            pltpu.sync_copy(rows_hbm.at[sl], hn_s.at[sl])

        plsc.subcore_barrier()

        def _idx(j):
            return src_v.at[pl.ds(j * _E_CHUNK, _E_CHUNK)]

        for b in range(_NBUF):
            pltpu.async_copy(hn_s.at[_idx(b)], rows_v.at[b], sems[b])

        def body(g, carry):
            for b in range(_NBUF):
                j = g * _NBUF + b
                pltpu.make_async_copy(hn_s.at[_idx(j)], rows_v.at[b], sems[b]).wait()
                pltpu.sync_copy(
                    rows_v.at[b],
                    acc.at[dst_v.at[pl.ds(j * _E_CHUNK, _E_CHUNK)]],
                    add=True,
                )
                nxt = j + _NBUF

                @pl.when(nxt < _CHUNKS_PER_TILE)
                def _refill():
                    pltpu.async_copy(hn_s.at[_idx(nxt)], rows_v.at[b], sems[b])

            return carry

        lax.fori_loop(0, _CHUNKS_PER_TILE // _NBUF, body, 0)
        plsc.subcore_barrier()
        _acc_writeback(acc, out0, out1, c, s)

    return edge_kernel


def _edge_kernel(src_flat, dst_flat, rows, zeros):
    return _make_edge_kernel()(src_flat, dst_flat, rows, zeros)


def _dense1_body(x_ref, w_ref, d0_ref, d1_ref, hn_ref, dis_ref):
    h = jnp.dot(x_ref[...], w_ref[...], preferred_element_type=jnp.float32)
    deg = d0_ref[...] + d1_ref[...] + 1.0
    dis = lax.rsqrt(deg)
    dis_ref[...] = dis
    hn_ref[...] = h * dis


def _dense1(x, W1, d0, d1):
    return pl.pallas_call(
        _dense1_body,
        grid=(_TC_GRID,),
        in_specs=[
            pl.BlockSpec((_BR, _D), lambda i: (i, 0)),
            pl.BlockSpec((_D, _H), lambda i: (0, 0)),
            pl.BlockSpec((_BR, _L), lambda i: (i, 0)),
            pl.BlockSpec((_BR, _L), lambda i: (i, 0)),
        ],
        out_specs=[
            pl.BlockSpec((_BR, _H), lambda i: (i, 0)),
            pl.BlockSpec((_BR, _L), lambda i: (i, 0)),
        ],
        out_shape=[
            jax.ShapeDtypeStruct((_N, _H), jnp.float32),
            jax.ShapeDtypeStruct((_N, _L), jnp.float32),
        ],
    )(x, W1, d0, d1)


def _dense2_body(a0_ref, a1_ref, hn1_ref, dis_ref, b1_ref, w2_ref, hn2_ref):
    pre = (a0_ref[...] + a1_ref[...] + hn1_ref[...]) * dis_ref[...] + b1_ref[...]
    a = jnp.maximum(pre, 0.0)
    h2 = jnp.dot(a, w2_ref[...], preferred_element_type=jnp.float32)
    hn2_ref[...] = h2 * dis_ref[...]


def _dense2(a0, a1, hn1, dis, b1, W2):
    return pl.pallas_call(
        _dense2_body,
        grid=(_TC_GRID,),
        in_specs=[
            pl.BlockSpec((_BR, _L), lambda i: (i, 0)),
            pl.BlockSpec((_BR, _L), lambda i: (i, 0)),
            pl.BlockSpec((_BR, _H), lambda i: (i, 0)),
            pl.BlockSpec((_BR, _L), lambda i: (i, 0)),
            pl.BlockSpec((1, _H), lambda i: (0, 0)),
            pl.BlockSpec((_H, _H), lambda i: (0, 0)),
        ],
        out_specs=pl.BlockSpec((_BR, _H), lambda i: (i, 0)),
        out_shape=jax.ShapeDtypeStruct((_N, _H), jnp.float32),
    )(a0, a1, hn1, dis, b1, W2)


def _final_body(a0_ref, a1_ref, hn2_ref, dis_ref, b2_ref, batch_ref, wc_ref,
                bc_ref, out_ref, psum, pcnt):
    i = pl.program_id(0)

    @pl.when(i == 0)
    def _init():
        psum[...] = jnp.zeros_like(psum)
        pcnt[...] = jnp.zeros_like(pcnt)

    out2 = (a0_ref[...] + a1_ref[...] + hn2_ref[...]) * dis_ref[...] + b2_ref[...]
    b = batch_ref[...]
    onehot = (b == lax.broadcasted_iota(jnp.int32, (1, _G), 1)).astype(jnp.float32)
    psum[...] += lax.dot_general(onehot, out2, (((0,), (0,)), ((), ())),
                                 preferred_element_type=jnp.float32)
    pcnt[...] += lax.dot_general(onehot, jnp.ones_like(out2), (((0,), (0,)), ((), ())),
                                 preferred_element_type=jnp.float32)

    @pl.when(i == _TC_GRID - 1)
    def _finish():
        pooled = psum[...] / jnp.maximum(pcnt[...], 1.0)
        logits = jnp.dot(pooled, wc_ref[...], preferred_element_type=jnp.float32)
        logits = logits + bc_ref[...]
        m = jnp.max(logits, axis=1, keepdims=True)
        e = jnp.exp(logits - m)
        lse = m + jnp.log(jnp.sum(e, axis=1, keepdims=True))
        out_ref[...] = logits - lse


def _final(a0, a1, hn2, dis, b2, batch2d, Wc, bc):
    return pl.pallas_call(
        _final_body,
        grid=(_TC_GRID,),
        in_specs=[
            pl.BlockSpec((_BR, _L), lambda i: (i, 0)),
            pl.BlockSpec((_BR, _L), lambda i: (i, 0)),
            pl.BlockSpec((_BR, _H), lambda i: (i, 0)),
            pl.BlockSpec((_BR, _L), lambda i: (i, 0)),
            pl.BlockSpec((1, _H), lambda i: (0, 0)),
            pl.BlockSpec((_BR, 1), lambda i: (i, 0)),
            pl.BlockSpec((_H, _CLS), lambda i: (0, 0)),
            pl.BlockSpec((1, _CLS), lambda i: (0, 0)),
        ],
        out_specs=pl.BlockSpec((_G, _CLS), lambda i: (0, 0)),
        out_shape=jax.ShapeDtypeStruct((_G, _CLS), jnp.float32),
        scratch_shapes=[
            pltpu.VMEM((_G, _H), jnp.float32),
            pltpu.VMEM((_G, _H), jnp.float32),
        ],
    )(a0, a1, hn2, dis, b2, batch2d, Wc, bc)


def kernel(x, edge_index, batch, W1, b1, W2, b2, Wc, bc):
    src_flat = jnp.concatenate(
        [edge_index[0].astype(jnp.int32), jnp.zeros((_E_PAD - _E,), jnp.int32)])
    dst_flat = jnp.concatenate(
        [edge_index[1].astype(jnp.int32), jnp.full((_E_PAD - _E,), _N, jnp.int32)])

    batch2d = batch.astype(jnp.int32).reshape(_N, 1)
    zeros = jnp.zeros((_ACC_PER_SUB, _L), jnp.float32)
    ones = jnp.ones((_E_CHUNK, _L), jnp.float32)

    d0, d1 = _deg_kernel(dst_flat, ones, zeros)
    hn1, dis = _dense1(x, W1, d0, d1)
    a0, a1 = _edge_kernel(src_flat, dst_flat, hn1, zeros)
    hn2 = _dense2(a0, a1, hn1, dis, b1.reshape(1, _H), W2)
    c0, c1 = _edge_kernel(src_flat, dst_flat, hn2, zeros)
    return _final(c0, c1, hn2, dis, b2.reshape(1, _H), batch2d, Wc,
                  bc.reshape(1, _CLS))


# speedup vs baseline: 1.1234x; 1.1234x over previous
import functools

import jax
import jax.numpy as jnp
from jax import lax
from jax.experimental import pallas as pl
from jax.experimental.pallas import tpu as pltpu
from jax.experimental.pallas import tpu_sc as plsc

_N = 10000
_E = 320000
_D = 128
_H = 16
_G = 64
_CLS = 2

_NC = 2
_NS = 16
_L = 16

_N_ACC = 10112
_ACC_PER_SUB = _N_ACC // _NS
_E_CHUNK = 128
_N_CHUNKS = _E // _E_CHUNK
_CHUNK_BASE = _N_CHUNKS // (_NC * _NS)
_CHUNK_EXTRA = _N_CHUNKS - _CHUNK_BASE * _NC * _NS
_MAX_CHUNKS = _CHUNK_BASE + 1
_TRIPS = (_MAX_CHUNKS + 3) // 4

_STG_BIG = 632
_STG_LAST = _N - _STG_BIG * (_NS - 1)

_BR = 2000
_TC_GRID = _N // _BR

_NBUF = 4


def _sc_mesh():
    return plsc.VectorSubcoreMesh(core_axis_name="c", subcore_axis_name="s")


def _tile_chunks(gtile):
    n = jnp.where(gtile < _CHUNK_EXTRA, _CHUNK_BASE + 1, _CHUNK_BASE)
    base = gtile * _CHUNK_BASE + jnp.minimum(gtile, _CHUNK_EXTRA)
    return n, base


def _fill(ref, rows, value):
    def body(i, carry):
        ref[i] = jnp.full((_L,), value, jnp.float32)
        return carry

    lax.fori_loop(0, rows, body, 0)


def _acc_writeback(acc, out0, out1, c, s):
    sl = pl.ds(s * _ACC_PER_SUB, _ACC_PER_SUB)

    @pl.when(c == 0)
    def _w0():
        pltpu.sync_copy(acc.at[sl], out0.at[sl])

    @pl.when(c == 1)
    def _w1():
        pltpu.sync_copy(acc.at[sl], out1.at[sl])


def _stage_indices(ei_hbm, row, base, n_chunks, idx_v):
    @pl.when(n_chunks == _CHUNK_BASE)
    def _cp_base():
        pltpu.sync_copy(
            ei_hbm.at[row, pl.ds(base * _E_CHUNK, _CHUNK_BASE * _E_CHUNK)],
            idx_v.at[pl.ds(0, _CHUNK_BASE * _E_CHUNK)],
        )

    @pl.when(n_chunks == _MAX_CHUNKS)
    def _cp_extra():
        pltpu.sync_copy(
            ei_hbm.at[row, pl.ds(base * _E_CHUNK, _MAX_CHUNKS * _E_CHUNK)],
            idx_v.at[pl.ds(0, _MAX_CHUNKS * _E_CHUNK)],
        )


@functools.cache
def _make_deg_kernel():
    @functools.partial(
        pl.kernel,
        out_type=(
            jax.ShapeDtypeStruct((_N_ACC, _L), jnp.float32),
            jax.ShapeDtypeStruct((_N_ACC, _L), jnp.float32),
        ),
        mesh=_sc_mesh(),
        scratch_types=[
            pltpu.VMEM((_MAX_CHUNKS * _E_CHUNK,), jnp.int32),
            pltpu.VMEM((_E_CHUNK, _L), jnp.float32),
            pltpu.VMEM((_ACC_PER_SUB, _L), jnp.float32),
            pltpu.VMEM_SHARED((_N_ACC, _L), jnp.float32),
        ],
        compiler_params=pltpu.CompilerParams(use_tc_tiling_on_sc=False),
    )
    def deg_kernel(ei_hbm, out0, out1, dst_v, ones_v, zeros_v, acc):
        c = lax.axis_index("c")
        s = lax.axis_index("s")
        gtile = c * _NS + s
        n_chunks, base = _tile_chunks(gtile)
        _stage_indices(ei_hbm, 1, base, n_chunks, dst_v)
        _fill(ones_v, _E_CHUNK, 1.0)
        _fill(zeros_v, _ACC_PER_SUB, 0.0)
        pltpu.sync_copy(zeros_v, acc.at[pl.ds(s * _ACC_PER_SUB, _ACC_PER_SUB)])
        plsc.subcore_barrier()

        def body(j, carry):
            pltpu.sync_copy(ones_v, acc.at[dst_v.at[pl.ds(j * _E_CHUNK, _E_CHUNK)]],
                            add=True)
            return carry

        lax.fori_loop(0, n_chunks, body, 0)
        plsc.subcore_barrier()
        _acc_writeback(acc, out0, out1, c, s)

    return deg_kernel


def _deg_kernel(edge_index):
    return _make_deg_kernel()(edge_index)


@functools.cache
def _make_edge_kernel():
    @functools.partial(
        pl.kernel,
        out_type=(
            jax.ShapeDtypeStruct((_N_ACC, _L), jnp.float32),
            jax.ShapeDtypeStruct((_N_ACC, _L), jnp.float32),
        ),
        mesh=_sc_mesh(),
        scratch_types=[
            pltpu.VMEM((_MAX_CHUNKS * _E_CHUNK,), jnp.int32),
            pltpu.VMEM((_MAX_CHUNKS * _E_CHUNK,), jnp.int32),
            pltpu.VMEM((_NBUF, _E_CHUNK, _L), jnp.float32),
            pltpu.VMEM((_ACC_PER_SUB, _L), jnp.float32),
        ]
        + [pltpu.SemaphoreType.DMA] * _NBUF
        + [
            pltpu.VMEM_SHARED((_N_ACC, _L), jnp.float32),
            pltpu.VMEM_SHARED((_N, _L), jnp.float32),
        ],
        compiler_params=pltpu.CompilerParams(use_tc_tiling_on_sc=False),
    )
    def edge_kernel(ei_hbm, rows_hbm, out0, out1,
                    src_v, dst_v, rows_v, zeros_v, sem0, sem1, sem2, sem3,
                    acc, hn_s):
        sems = (sem0, sem1, sem2, sem3)
        c = lax.axis_index("c")
        s = lax.axis_index("s")
        gtile = c * _NS + s
        n_chunks, base = _tile_chunks(gtile)
        _stage_indices(ei_hbm, 0, base, n_chunks, src_v)
        _stage_indices(ei_hbm, 1, base, n_chunks, dst_v)
        _fill(zeros_v, _ACC_PER_SUB, 0.0)
        pltpu.sync_copy(zeros_v, acc.at[pl.ds(s * _ACC_PER_SUB, _ACC_PER_SUB)])

        @pl.when(s < _NS - 1)
        def _stage_big():
            sl = pl.ds(s * _STG_BIG, _STG_BIG)
            pltpu.sync_copy(rows_hbm.at[sl], hn_s.at[sl])

        @pl.when(s == _NS - 1)
        def _stage_last():
            sl = pl.ds((_NS - 1) * _STG_BIG, _STG_LAST)
            pltpu.sync_copy(rows_hbm.at[sl], hn_s.at[sl])

        plsc.subcore_barrier()

        def _idx(j):
            return src_v.at[pl.ds(j * _E_CHUNK, _E_CHUNK)]

        for b in range(_NBUF):
            pltpu.async_copy(hn_s.at[_idx(b)], rows_v.at[b], sems[b])

        def body(g, carry):
            for b in range(_NBUF):
                j = g * _NBUF + b

                @pl.when(j < n_chunks)
                def _do():
                    pltpu.make_async_copy(hn_s.at[_idx(j)], rows_v.at[b],
                                          sems[b]).wait()
                    pltpu.sync_copy(
                        rows_v.at[b],
                        acc.at[dst_v.at[pl.ds(j * _E_CHUNK, _E_CHUNK)]],
                        add=True,
                    )
                    nxt = j + _NBUF

                    @pl.when(nxt < n_chunks)
                    def _refill():
                        pltpu.async_copy(hn_s.at[_idx(nxt)], rows_v.at[b],
                                         sems[b])

            return carry

        lax.fori_loop(0, _TRIPS, body, 0)
        plsc.subcore_barrier()
        _acc_writeback(acc, out0, out1, c, s)

    return edge_kernel


def _edge_kernel(edge_index, rows):
    return _make_edge_kernel()(edge_index, rows)


def _dense1_body(x_ref, w_ref, d0_ref, d1_ref, hn_ref, dis_ref):
    h = jnp.dot(x_ref[...], w_ref[...], preferred_element_type=jnp.float32)
    deg = d0_ref[...] + d1_ref[...] + 1.0
    dis = lax.rsqrt(deg)
    dis_ref[...] = dis
    hn_ref[...] = h * dis


def _dense1(x, W1, d0, d1):
    return pl.pallas_call(
        _dense1_body,
        grid=(_TC_GRID,),
        in_specs=[
            pl.BlockSpec((_BR, _D), lambda i: (i, 0)),
            pl.BlockSpec((_D, _H), lambda i: (0, 0)),
            pl.BlockSpec((_BR, _L), lambda i: (i, 0)),
            pl.BlockSpec((_BR, _L), lambda i: (i, 0)),
        ],
        out_specs=[
            pl.BlockSpec((_BR, _H), lambda i: (i, 0)),
            pl.BlockSpec((_BR, _L), lambda i: (i, 0)),
        ],
        out_shape=[
            jax.ShapeDtypeStruct((_N, _H), jnp.float32),
            jax.ShapeDtypeStruct((_N, _L), jnp.float32),
        ],
    )(x, W1, d0, d1)


def _dense2_body(a0_ref, a1_ref, hn1_ref, dis_ref, b1_ref, w2_ref, hn2_ref):
    pre = (a0_ref[...] + a1_ref[...] + hn1_ref[...]) * dis_ref[...] + b1_ref[...]
    a = jnp.maximum(pre, 0.0)
    h2 = jnp.dot(a, w2_ref[...], preferred_element_type=jnp.float32)
    hn2_ref[...] = h2 * dis_ref[...]


def _dense2(a0, a1, hn1, dis, b1, W2):
    return pl.pallas_call(
        _dense2_body,
        grid=(_TC_GRID,),
        in_specs=[
            pl.BlockSpec((_BR, _L), lambda i: (i, 0)),
            pl.BlockSpec((_BR, _L), lambda i: (i, 0)),
            pl.BlockSpec((_BR, _H), lambda i: (i, 0)),
            pl.BlockSpec((_BR, _L), lambda i: (i, 0)),
            pl.BlockSpec((1, _H), lambda i: (0, 0)),
            pl.BlockSpec((_H, _H), lambda i: (0, 0)),
        ],
        out_specs=pl.BlockSpec((_BR, _H), lambda i: (i, 0)),
        out_shape=jax.ShapeDtypeStruct((_N, _H), jnp.float32),
    )(a0, a1, hn1, dis, b1, W2)


def _final_body(a0_ref, a1_ref, hn2_ref, dis_ref, b2_ref, batch_ref, wc_ref,
                bc_ref, out_ref, psum, pcnt):
    i = pl.program_id(0)

    @pl.when(i == 0)
    def _init():
        psum[...] = jnp.zeros_like(psum)
        pcnt[...] = jnp.zeros_like(pcnt)

    out2 = (a0_ref[...] + a1_ref[...] + hn2_ref[...]) * dis_ref[...] + b2_ref[...]
    b = batch_ref[...]
    onehot = (b == lax.broadcasted_iota(jnp.int32, (1, _G), 1)).astype(jnp.float32)
    psum[...] += lax.dot_general(onehot, out2, (((0,), (0,)), ((), ())),
                                 preferred_element_type=jnp.float32)
    pcnt[...] += lax.dot_general(onehot, jnp.ones_like(out2), (((0,), (0,)), ((), ())),
                                 preferred_element_type=jnp.float32)

    @pl.when(i == _TC_GRID - 1)
    def _finish():
        pooled = psum[...] / jnp.maximum(pcnt[...], 1.0)
        logits = jnp.dot(pooled, wc_ref[...], preferred_element_type=jnp.float32)
        logits = logits + bc_ref[...]
        m = jnp.max(logits, axis=1, keepdims=True)
        e = jnp.exp(logits - m)
        lse = m + jnp.log(jnp.sum(e, axis=1, keepdims=True))
        out_ref[...] = logits - lse


def _final(a0, a1, hn2, dis, b2, batch2d, Wc, bc):
    return pl.pallas_call(
        _final_body,
        grid=(_TC_GRID,),
        in_specs=[
            pl.BlockSpec((_BR, _L), lambda i: (i, 0)),
            pl.BlockSpec((_BR, _L), lambda i: (i, 0)),
            pl.BlockSpec((_BR, _H), lambda i: (i, 0)),
            pl.BlockSpec((_BR, _L), lambda i: (i, 0)),
            pl.BlockSpec((1, _H), lambda i: (0, 0)),
            pl.BlockSpec((_BR, 1), lambda i: (i, 0)),
            pl.BlockSpec((_H, _CLS), lambda i: (0, 0)),
            pl.BlockSpec((1, _CLS), lambda i: (0, 0)),
        ],
        out_specs=pl.BlockSpec((_G, _CLS), lambda i: (0, 0)),
        out_shape=jax.ShapeDtypeStruct((_G, _CLS), jnp.float32),
        scratch_shapes=[
            pltpu.VMEM((_G, _H), jnp.float32),
            pltpu.VMEM((_G, _H), jnp.float32),
        ],
    )(a0, a1, hn2, dis, b2, batch2d, Wc, bc)


def kernel(x, edge_index, batch, W1, b1, W2, b2, Wc, bc):
    ei = edge_index.astype(jnp.int32)
    batch2d = batch.astype(jnp.int32).reshape(_N, 1)

    d0, d1 = _deg_kernel(ei)
    hn1, dis = _dense1(x, W1, d0, d1)
    a0, a1 = _edge_kernel(ei, hn1)
    hn2 = _dense2(a0, a1, hn1, dis, b1.reshape(1, _H), W2)
    c0, c1 = _edge_kernel(ei, hn2)
    return _final(c0, c1, hn2, dis, b2.reshape(1, _H), batch2d, Wc,
                  bc.reshape(1, _CLS))


# speedup vs baseline: 1.6446x; 1.4640x over previous
import functools

import jax
import jax.numpy as jnp
from jax import lax
from jax.experimental import pallas as pl
from jax.experimental.pallas import tpu as pltpu
from jax.experimental.pallas import tpu_sc as plsc

_N = 10000
_E = 320000
_D = 128
_H = 16
_G = 64
_CLS = 2

_NC = 2
_NS = 16
_L = 16

_N_ACC = 10112
_ACC_PER_SUB = _N_ACC // _NS
_E_CHUNK = 128
_N_CHUNKS = _E // _E_CHUNK
_CHUNK_BASE = _N_CHUNKS // (_NC * _NS)
_CHUNK_EXTRA = _N_CHUNKS - _CHUNK_BASE * _NC * _NS
_MAX_CHUNKS = _CHUNK_BASE + 1
_TRIPS = (_MAX_CHUNKS + 3) // 4

_N_VIEW = _N_ACC // 8
_BRV = 632
_BRL = _BRV * 8
_TC_GRID = _N_VIEW // _BRV

_NBUF = 4


def _sc_mesh():
    return plsc.VectorSubcoreMesh(core_axis_name="c", subcore_axis_name="s")


def _tile_chunks(gtile):
    n = jnp.where(gtile < _CHUNK_EXTRA, _CHUNK_BASE + 1, _CHUNK_BASE)
    base = gtile * _CHUNK_BASE + jnp.minimum(gtile, _CHUNK_EXTRA)
    return n, base


def _fill(ref, rows, value):
    def body(i, carry):
        ref[i] = jnp.full((_L,), value, jnp.float32)
        return carry

    lax.fori_loop(0, rows, body, 0)


def _acc_writeback(acc, out0, out1, c, s):
    sl = pl.ds(s * _ACC_PER_SUB, _ACC_PER_SUB)

    @pl.when(c == 0)
    def _w0():
        pltpu.sync_copy(acc.at[sl], out0.at[sl])

    @pl.when(c == 1)
    def _w1():
        pltpu.sync_copy(acc.at[sl], out1.at[sl])


def _stage_indices(ei_hbm, row, base, n_chunks, idx_v):
    @pl.when(n_chunks == _CHUNK_BASE)
    def _cp_base():
        pltpu.sync_copy(
            ei_hbm.at[row, pl.ds(base * _E_CHUNK, _CHUNK_BASE * _E_CHUNK)],
            idx_v.at[pl.ds(0, _CHUNK_BASE * _E_CHUNK)],
        )

    @pl.when(n_chunks == _MAX_CHUNKS)
    def _cp_extra():
        pltpu.sync_copy(
            ei_hbm.at[row, pl.ds(base * _E_CHUNK, _MAX_CHUNKS * _E_CHUNK)],
            idx_v.at[pl.ds(0, _MAX_CHUNKS * _E_CHUNK)],
        )


@functools.cache
def _make_deg_kernel():
    @functools.partial(
        pl.kernel,
        out_type=(
            jax.ShapeDtypeStruct((_N_ACC, _L), jnp.float32),
            jax.ShapeDtypeStruct((_N_ACC, _L), jnp.float32),
        ),
        mesh=_sc_mesh(),
        scratch_types=[
            pltpu.VMEM((_MAX_CHUNKS * _E_CHUNK,), jnp.int32),
            pltpu.VMEM((_E_CHUNK, _L), jnp.float32),
            pltpu.VMEM((_ACC_PER_SUB, _L), jnp.float32),
            pltpu.VMEM_SHARED((_N_ACC, _L), jnp.float32),
        ],
        compiler_params=pltpu.CompilerParams(use_tc_tiling_on_sc=False),
    )
    def deg_kernel(ei_hbm, out0, out1, dst_v, ones_v, zeros_v, acc):
        c = lax.axis_index("c")
        s = lax.axis_index("s")
        gtile = c * _NS + s
        n_chunks, base = _tile_chunks(gtile)
        _stage_indices(ei_hbm, 1, base, n_chunks, dst_v)
        _fill(ones_v, _E_CHUNK, 1.0)
        _fill(zeros_v, _ACC_PER_SUB, 0.0)
        pltpu.sync_copy(zeros_v, acc.at[pl.ds(s * _ACC_PER_SUB, _ACC_PER_SUB)])
        plsc.subcore_barrier()

        def body(j, carry):
            pltpu.sync_copy(ones_v, acc.at[dst_v.at[pl.ds(j * _E_CHUNK, _E_CHUNK)]],
                            add=True)
            return carry

        lax.fori_loop(0, n_chunks, body, 0)
        plsc.subcore_barrier()
        _acc_writeback(acc, out0, out1, c, s)

    return deg_kernel


def _deg_kernel(edge_index):
    return _make_deg_kernel()(edge_index)


@functools.cache
def _make_edge_kernel():
    @functools.partial(
        pl.kernel,
        out_type=(
            jax.ShapeDtypeStruct((_N_ACC, _L), jnp.float32),
            jax.ShapeDtypeStruct((_N_ACC, _L), jnp.float32),
        ),
        mesh=_sc_mesh(),
        scratch_types=[
            pltpu.VMEM((_MAX_CHUNKS * _E_CHUNK,), jnp.int32),
            pltpu.VMEM((_MAX_CHUNKS * _E_CHUNK,), jnp.int32),
            pltpu.VMEM((_NBUF, _E_CHUNK, _L), jnp.float32),
            pltpu.VMEM((_ACC_PER_SUB, _L), jnp.float32),
        ]
        + [pltpu.SemaphoreType.DMA] * _NBUF
        + [
            pltpu.VMEM_SHARED((_N_ACC, _L), jnp.float32),
            pltpu.VMEM_SHARED((_N_ACC, _L), jnp.float32),
        ],
        compiler_params=pltpu.CompilerParams(use_tc_tiling_on_sc=False),
    )
    def edge_kernel(ei_hbm, rows_hbm, out0, out1,
                    src_v, dst_v, rows_v, zeros_v, sem0, sem1, sem2, sem3,
                    acc, hn_s):
        sems = (sem0, sem1, sem2, sem3)
        c = lax.axis_index("c")
        s = lax.axis_index("s")
        gtile = c * _NS + s
        n_chunks, base = _tile_chunks(gtile)
        _stage_indices(ei_hbm, 0, base, n_chunks, src_v)
        _stage_indices(ei_hbm, 1, base, n_chunks, dst_v)
        _fill(zeros_v, _ACC_PER_SUB, 0.0)
        pltpu.sync_copy(zeros_v, acc.at[pl.ds(s * _ACC_PER_SUB, _ACC_PER_SUB)])
        stg = pl.ds(s * _ACC_PER_SUB, _ACC_PER_SUB)
        pltpu.sync_copy(rows_hbm.at[stg], hn_s.at[stg])
        plsc.subcore_barrier()

        def _idx(j):
            return src_v.at[pl.ds(j * _E_CHUNK, _E_CHUNK)]

        for b in range(_NBUF):
            pltpu.async_copy(hn_s.at[_idx(b)], rows_v.at[b], sems[b])

        def body(g, carry):
            for b in range(_NBUF):
                j = g * _NBUF + b

                @pl.when(j < n_chunks)
                def _do():
                    pltpu.make_async_copy(hn_s.at[_idx(j)], rows_v.at[b],
                                          sems[b]).wait()
                    pltpu.sync_copy(
                        rows_v.at[b],
                        acc.at[dst_v.at[pl.ds(j * _E_CHUNK, _E_CHUNK)]],
                        add=True,
                    )
                    nxt = j + _NBUF

                    @pl.when(nxt < n_chunks)
                    def _refill():
                        pltpu.async_copy(hn_s.at[_idx(nxt)], rows_v.at[b],
                                         sems[b])

            return carry

        lax.fori_loop(0, _TRIPS, body, 0)
        plsc.subcore_barrier()
        _acc_writeback(acc, out0, out1, c, s)

    return edge_kernel


def _edge_kernel(edge_index, rows):
    return _make_edge_kernel()(edge_index, rows)


def _dense1_body(xv_ref, w1v_ref, d0_ref, d1_ref, hn_ref, dis_ref):
    hv = jnp.dot(xv_ref[...], w1v_ref[...], preferred_element_type=jnp.float32)
    deg = d0_ref[...] + d1_ref[...] + 1.0
    dis = lax.rsqrt(deg)
    dis_ref[...] = dis
    hn_ref[...] = hv * dis


def _dense1(xv, W1v, d0v, d1v):
    return pl.pallas_call(
        _dense1_body,
        grid=(_TC_GRID,),
        in_specs=[
            pl.BlockSpec((_BRV, 8 * _D), lambda i: (i, 0)),
            pl.BlockSpec((8 * _D, 8 * _H), lambda i: (0, 0)),
            pl.BlockSpec((_BRV, 8 * _L), lambda i: (i, 0)),
            pl.BlockSpec((_BRV, 8 * _L), lambda i: (i, 0)),
        ],
        out_specs=[
            pl.BlockSpec((_BRV, 8 * _H), lambda i: (i, 0)),
            pl.BlockSpec((_BRV, 8 * _L), lambda i: (i, 0)),
        ],
        out_shape=[
            jax.ShapeDtypeStruct((_N_VIEW, 8 * _H), jnp.float32),
            jax.ShapeDtypeStruct((_N_VIEW, 8 * _L), jnp.float32),
        ],
    )(xv, W1v, d0v, d1v)


def _dense2_body(a0_ref, a1_ref, hn1_ref, dis_ref, b1_ref, w2v_ref, hn2_ref):
    pre = (a0_ref[...] + a1_ref[...] + hn1_ref[...]) * dis_ref[...] + b1_ref[...]
    a = jnp.maximum(pre, 0.0)
    h2v = jnp.dot(a, w2v_ref[...], preferred_element_type=jnp.float32)
    hn2_ref[...] = h2v * dis_ref[...]


def _dense2(a0v, a1v, hn1v, disv, b1v, W2v):
    return pl.pallas_call(
        _dense2_body,
        grid=(_TC_GRID,),
        in_specs=[
            pl.BlockSpec((_BRV, 8 * _L), lambda i: (i, 0)),
            pl.BlockSpec((_BRV, 8 * _L), lambda i: (i, 0)),
            pl.BlockSpec((_BRV, 8 * _H), lambda i: (i, 0)),
            pl.BlockSpec((_BRV, 8 * _L), lambda i: (i, 0)),
            pl.BlockSpec((1, 8 * _H), lambda i: (0, 0)),
            pl.BlockSpec((8 * _H, 8 * _H), lambda i: (0, 0)),
        ],
        out_specs=pl.BlockSpec((_BRV, 8 * _H), lambda i: (i, 0)),
        out_shape=jax.ShapeDtypeStruct((_N_VIEW, 8 * _H), jnp.float32),
    )(a0v, a1v, hn1v, disv, b1v, W2v)


def _final_body(a0_ref, a1_ref, hn2_ref, dis_ref, b2_ref, batch_ref, wc_ref,
                bc_ref, out_ref, psum, pcnt):
    i = pl.program_id(0)

    @pl.when(i == 0)
    def _init():
        psum[...] = jnp.zeros_like(psum)
        pcnt[...] = jnp.zeros_like(pcnt)

    out2v = (a0_ref[...] + a1_ref[...] + hn2_ref[...]) * dis_ref[...] + b2_ref[...]
    b = batch_ref[...]
    iota_g = lax.broadcasted_iota(jnp.int32, (1, _G), 1)
    acc_s = psum[...]
    acc_c = pcnt[...]
    ones16 = jnp.ones((_BRV, _H), jnp.float32)
    for k in range(8):
        onehot = (b[:, k:k + 1] == iota_g).astype(jnp.float32)
        sl = out2v[:, k * _H:(k + 1) * _H]
        acc_s += lax.dot_general(onehot, sl, (((0,), (0,)), ((), ())),
                                 preferred_element_type=jnp.float32)
        acc_c += lax.dot_general(onehot, ones16, (((0,), (0,)), ((), ())),
                                 preferred_element_type=jnp.float32)
    psum[...] = acc_s
    pcnt[...] = acc_c

    @pl.when(i == _TC_GRID - 1)
    def _finish():
        pooled = psum[...] / jnp.maximum(pcnt[...], 1.0)
        logits = jnp.dot(pooled, wc_ref[...], preferred_element_type=jnp.float32)
        logits = logits + bc_ref[...]
        m = jnp.max(logits, axis=1, keepdims=True)
        e = jnp.exp(logits - m)
        lse = m + jnp.log(jnp.sum(e, axis=1, keepdims=True))
        out_ref[...] = logits - lse


def _final(a0v, a1v, hn2v, disv, b2v, batchv, Wc, bc):
    return pl.pallas_call(
        _final_body,
        grid=(_TC_GRID,),
        in_specs=[
            pl.BlockSpec((_BRV, 8 * _L), lambda i: (i, 0)),
            pl.BlockSpec((_BRV, 8 * _L), lambda i: (i, 0)),
            pl.BlockSpec((_BRV, 8 * _H), lambda i: (i, 0)),
            pl.BlockSpec((_BRV, 8 * _L), lambda i: (i, 0)),
            pl.BlockSpec((1, 8 * _H), lambda i: (0, 0)),
            pl.BlockSpec((_BRV, 8), lambda i: (i, 0)),
            pl.BlockSpec((_H, _CLS), lambda i: (0, 0)),
            pl.BlockSpec((1, _CLS), lambda i: (0, 0)),
        ],
        out_specs=pl.BlockSpec((_G, _CLS), lambda i: (0, 0)),
        out_shape=jax.ShapeDtypeStruct((_G, _CLS), jnp.float32),
        scratch_shapes=[
            pltpu.VMEM((_G, _H), jnp.float32),
            pltpu.VMEM((_G, _H), jnp.float32),
        ],
    )(a0v, a1v, hn2v, disv, b2v, batchv, Wc, bc)


def _view(p):
    return p.reshape(_N_VIEW, 8 * _L)


def _blockdiag8(W):
    k, m = W.shape
    out = jnp.zeros((8 * k, 8 * m), W.dtype)
    for i in range(8):
        out = lax.dynamic_update_slice(out, W, (i * k, i * m))
    return out


def kernel(x, edge_index, batch, W1, b1, W2, b2, Wc, bc):
    ei = edge_index.astype(jnp.int32)
    xv = jnp.concatenate(
        [x, jnp.zeros((_N_ACC - _N, _D), jnp.float32)]).reshape(_N_VIEW, 8 * _D)
    batchv = jnp.concatenate(
        [batch.astype(jnp.int32), jnp.full((_N_ACC - _N,), _G, jnp.int32)]
    ).reshape(_N_VIEW, 8)
    b1v = jnp.tile(b1, 8).reshape(1, 8 * _H)
    b2v = jnp.tile(b2, 8).reshape(1, 8 * _H)
    W1v = _blockdiag8(W1)
    W2v = _blockdiag8(W2)

    d0, d1 = _deg_kernel(ei)
    hn1v, disv = _dense1(xv, W1v, _view(d0), _view(d1))
    a0, a1 = _edge_kernel(ei, hn1v.reshape(_N_ACC, _L))
    hn2v = _dense2(_view(a0), _view(a1), hn1v, disv, b1v, W2v)
    c0, c1 = _edge_kernel(ei, hn2v.reshape(_N_ACC, _L))
    return _final(_view(c0), _view(c1), hn2v, disv, b2v, batchv, Wc,
                  bc.reshape(1, _CLS))


# speedup vs baseline: 1.7653x; 1.0734x over previous
import functools

import jax
import jax.numpy as jnp
from jax import lax
from jax.experimental import pallas as pl
from jax.experimental.pallas import tpu as pltpu
from jax.experimental.pallas import tpu_sc as plsc

_N = 10000
_E = 320000
_D = 128
_H = 16
_G = 64
_CLS = 2

_NC = 2
_NS = 16
_L = 16

_N_ACC = 10112
_ACC_PER_SUB = _N_ACC // _NS
_E_CHUNK = 128
_N_CHUNKS = _E // _E_CHUNK
_CHUNK_BASE = _N_CHUNKS // (_NC * _NS)
_CHUNK_EXTRA = _N_CHUNKS - _CHUNK_BASE * _NC * _NS
_MAX_CHUNKS = _CHUNK_BASE + 1
_TRIPS = (_MAX_CHUNKS + 7) // 8

_N_VIEW = _N_ACC // 8
_BRV = 632
_BRL = _BRV * 8
_TC_GRID = _N_VIEW // _BRV

_NBUF = 8
_GLEAD = 4


def _sc_mesh():
    return plsc.VectorSubcoreMesh(core_axis_name="c", subcore_axis_name="s")


def _tile_chunks(gtile):
    n = jnp.where(gtile < _CHUNK_EXTRA, _CHUNK_BASE + 1, _CHUNK_BASE)
    base = gtile * _CHUNK_BASE + jnp.minimum(gtile, _CHUNK_EXTRA)
    return n, base


def _fill(ref, rows, value):
    def body(i, carry):
        ref[i] = jnp.full((_L,), value, jnp.float32)
        return carry

    lax.fori_loop(0, rows, body, 0)


def _acc_writeback(acc, out0, out1, c, s):
    sl = pl.ds(s * _ACC_PER_SUB, _ACC_PER_SUB)

    @pl.when(c == 0)
    def _w0():
        pltpu.sync_copy(acc.at[sl], out0.at[sl])

    @pl.when(c == 1)
    def _w1():
        pltpu.sync_copy(acc.at[sl], out1.at[sl])


def _stage_indices(ei_hbm, row, base, n_chunks, idx_v):
    @pl.when(n_chunks == _CHUNK_BASE)
    def _cp_base():
        pltpu.sync_copy(
            ei_hbm.at[row, pl.ds(base * _E_CHUNK, _CHUNK_BASE * _E_CHUNK)],
            idx_v.at[pl.ds(0, _CHUNK_BASE * _E_CHUNK)],
        )

    @pl.when(n_chunks == _MAX_CHUNKS)
    def _cp_extra():
        pltpu.sync_copy(
            ei_hbm.at[row, pl.ds(base * _E_CHUNK, _MAX_CHUNKS * _E_CHUNK)],
            idx_v.at[pl.ds(0, _MAX_CHUNKS * _E_CHUNK)],
        )


@functools.cache
def _make_deg_kernel():
    @functools.partial(
        pl.kernel,
        out_type=(
            jax.ShapeDtypeStruct((_N_ACC, _L), jnp.float32),
            jax.ShapeDtypeStruct((_N_ACC, _L), jnp.float32),
        ),
        mesh=_sc_mesh(),
        scratch_types=[
            pltpu.VMEM((_MAX_CHUNKS * _E_CHUNK,), jnp.int32),
            pltpu.VMEM((_E_CHUNK, _L), jnp.float32),
            pltpu.VMEM((_ACC_PER_SUB, _L), jnp.float32),
            pltpu.VMEM_SHARED((_N_ACC, _L), jnp.float32),
        ],
        compiler_params=pltpu.CompilerParams(use_tc_tiling_on_sc=False),
    )
    def deg_kernel(ei_hbm, out0, out1, dst_v, ones_v, zeros_v, acc):
        c = lax.axis_index("c")
        s = lax.axis_index("s")
        gtile = c * _NS + s
        n_chunks, base = _tile_chunks(gtile)
        _stage_indices(ei_hbm, 1, base, n_chunks, dst_v)
        _fill(ones_v, _E_CHUNK, 1.0)
        _fill(zeros_v, _ACC_PER_SUB, 0.0)
        pltpu.sync_copy(zeros_v, acc.at[pl.ds(s * _ACC_PER_SUB, _ACC_PER_SUB)])
        plsc.subcore_barrier()

        def body(j, carry):
            pltpu.sync_copy(ones_v, acc.at[dst_v.at[pl.ds(j * _E_CHUNK, _E_CHUNK)]],
                            add=True)
            return carry

        lax.fori_loop(0, n_chunks, body, 0)
        plsc.subcore_barrier()
        _acc_writeback(acc, out0, out1, c, s)

    return deg_kernel


def _deg_kernel(edge_index):
    return _make_deg_kernel()(edge_index)


@functools.cache
def _make_edge_kernel():
    @functools.partial(
        pl.kernel,
        out_type=(
            jax.ShapeDtypeStruct((_N_ACC, _L), jnp.float32),
            jax.ShapeDtypeStruct((_N_ACC, _L), jnp.float32),
        ),
        mesh=_sc_mesh(),
        scratch_types=[
            pltpu.VMEM((_MAX_CHUNKS * _E_CHUNK,), jnp.int32),
            pltpu.VMEM((_MAX_CHUNKS * _E_CHUNK,), jnp.int32),
            pltpu.VMEM((_NBUF, _E_CHUNK, _L), jnp.float32),
            pltpu.VMEM((_ACC_PER_SUB, _L), jnp.float32),
        ]
        + [pltpu.SemaphoreType.DMA] * (2 * _NBUF)
        + [
            pltpu.VMEM_SHARED((_N_ACC, _L), jnp.float32),
            pltpu.VMEM_SHARED((_N_ACC, _L), jnp.float32),
        ],
        compiler_params=pltpu.CompilerParams(use_tc_tiling_on_sc=False),
    )
    def edge_kernel(ei_hbm, rows_hbm, out0, out1,
                    src_v, dst_v, rows_v, zeros_v,
                    g0, g1, g2, g3, g4, g5, g6, g7,
                    s0, s1, s2, s3, s4, s5, s6, s7,
                    acc, hn_s):
        gsems = (g0, g1, g2, g3, g4, g5, g6, g7)
        ssems = (s0, s1, s2, s3, s4, s5, s6, s7)
        c = lax.axis_index("c")
        s = lax.axis_index("s")
        gtile = c * _NS + s
        n_chunks, base = _tile_chunks(gtile)
        _stage_indices(ei_hbm, 0, base, n_chunks, src_v)
        _stage_indices(ei_hbm, 1, base, n_chunks, dst_v)
        _fill(zeros_v, _ACC_PER_SUB, 0.0)
        pltpu.sync_copy(zeros_v, acc.at[pl.ds(s * _ACC_PER_SUB, _ACC_PER_SUB)])
        stg = pl.ds(s * _ACC_PER_SUB, _ACC_PER_SUB)
        pltpu.sync_copy(rows_hbm.at[stg], hn_s.at[stg])
        plsc.subcore_barrier()

        def _sidx(j):
            return src_v.at[pl.ds(j * _E_CHUNK, _E_CHUNK)]

        def _didx(j):
            return dst_v.at[pl.ds(j * _E_CHUNK, _E_CHUNK)]

        for b in range(_GLEAD):
            pltpu.async_copy(hn_s.at[_sidx(b)], rows_v.at[b], gsems[b])

        def body(g, carry):
            for u in range(_NBUF):
                j = g * _NBUF + u

                @pl.when(j < n_chunks)
                def _slot():
                    pltpu.make_async_copy(hn_s.at[_sidx(j)], rows_v.at[u],
                                          gsems[u]).wait()
                    pltpu.async_copy(rows_v.at[u], acc.at[_didx(j)], ssems[u],
                                     add=True)
                    tgt = j + _GLEAD
                    tb = (u + _GLEAD) % _NBUF

                    @pl.when(tgt < n_chunks)
                    def _prefetch():
                        @pl.when(tgt >= _NBUF)
                        def _drain_prev():
                            pltpu.make_async_copy(rows_v.at[tb],
                                                  acc.at[_didx(0)],
                                                  ssems[tb]).wait()

                        pltpu.async_copy(hn_s.at[_sidx(tgt)], rows_v.at[tb],
                                         gsems[tb])

            return carry

        lax.fori_loop(0, _TRIPS, body, 0)
        for b in range(_NBUF):
            pltpu.make_async_copy(rows_v.at[b], acc.at[_didx(0)],
                                  ssems[b]).wait()
        plsc.subcore_barrier()
        _acc_writeback(acc, out0, out1, c, s)

    return edge_kernel


def _edge_kernel(edge_index, rows):
    return _make_edge_kernel()(edge_index, rows)


def _dense1_body(xv_ref, w1v_ref, d0_ref, d1_ref, hn_ref, dis_ref):
    hv = jnp.dot(xv_ref[...], w1v_ref[...], preferred_element_type=jnp.float32)
    deg = d0_ref[...] + d1_ref[...] + 1.0
    dis = lax.rsqrt(deg)
    dis_ref[...] = dis
    hn_ref[...] = hv * dis


def _dense1(xv, W1v, d0v, d1v):
    return pl.pallas_call(
        _dense1_body,
        grid=(_TC_GRID,),
        in_specs=[
            pl.BlockSpec((_BRV, 8 * _D), lambda i: (i, 0)),
            pl.BlockSpec((8 * _D, 8 * _H), lambda i: (0, 0)),
            pl.BlockSpec((_BRV, 8 * _L), lambda i: (i, 0)),
            pl.BlockSpec((_BRV, 8 * _L), lambda i: (i, 0)),
        ],
        out_specs=[
            pl.BlockSpec((_BRV, 8 * _H), lambda i: (i, 0)),
            pl.BlockSpec((_BRV, 8 * _L), lambda i: (i, 0)),
        ],
        out_shape=[
            jax.ShapeDtypeStruct((_N_VIEW, 8 * _H), jnp.float32),
            jax.ShapeDtypeStruct((_N_VIEW, 8 * _L), jnp.float32),
        ],
    )(xv, W1v, d0v, d1v)


def _dense2_body(a0_ref, a1_ref, hn1_ref, dis_ref, b1_ref, w2v_ref, hn2_ref):
    pre = (a0_ref[...] + a1_ref[...] + hn1_ref[...]) * dis_ref[...] + b1_ref[...]
    a = jnp.maximum(pre, 0.0)
    h2v = jnp.dot(a, w2v_ref[...], preferred_element_type=jnp.float32)
    hn2_ref[...] = h2v * dis_ref[...]


def _dense2(a0v, a1v, hn1v, disv, b1v, W2v):
    return pl.pallas_call(
        _dense2_body,
        grid=(_TC_GRID,),
        in_specs=[
            pl.BlockSpec((_BRV, 8 * _L), lambda i: (i, 0)),
            pl.BlockSpec((_BRV, 8 * _L), lambda i: (i, 0)),
            pl.BlockSpec((_BRV, 8 * _H), lambda i: (i, 0)),
            pl.BlockSpec((_BRV, 8 * _L), lambda i: (i, 0)),
            pl.BlockSpec((1, 8 * _H), lambda i: (0, 0)),
            pl.BlockSpec((8 * _H, 8 * _H), lambda i: (0, 0)),
        ],
        out_specs=pl.BlockSpec((_BRV, 8 * _H), lambda i: (i, 0)),
        out_shape=jax.ShapeDtypeStruct((_N_VIEW, 8 * _H), jnp.float32),
    )(a0v, a1v, hn1v, disv, b1v, W2v)


def _final_body(a0_ref, a1_ref, hn2_ref, dis_ref, b2_ref, batch_ref, wc_ref,
                bc_ref, out_ref, psum, pcnt):
    i = pl.program_id(0)

    @pl.when(i == 0)
    def _init():
        psum[...] = jnp.zeros_like(psum)
        pcnt[...] = jnp.zeros_like(pcnt)

    out2v = (a0_ref[...] + a1_ref[...] + hn2_ref[...]) * dis_ref[...] + b2_ref[...]
    b = batch_ref[...]
    iota_g = lax.broadcasted_iota(jnp.int32, (1, _G), 1)
    acc_s = psum[...]
    acc_c = pcnt[...]
    ones16 = jnp.ones((_BRV, _H), jnp.float32)
    for k in range(8):
        onehot = (b[:, k:k + 1] == iota_g).astype(jnp.float32)
        sl = out2v[:, k * _H:(k + 1) * _H]
        acc_s += lax.dot_general(onehot, sl, (((0,), (0,)), ((), ())),
                                 preferred_element_type=jnp.float32)
        acc_c += lax.dot_general(onehot, ones16, (((0,), (0,)), ((), ())),
                                 preferred_element_type=jnp.float32)
    psum[...] = acc_s
    pcnt[...] = acc_c

    @pl.when(i == _TC_GRID - 1)
    def _finish():
        pooled = psum[...] / jnp.maximum(pcnt[...], 1.0)
        logits = jnp.dot(pooled, wc_ref[...], preferred_element_type=jnp.float32)
        logits = logits + bc_ref[...]
        m = jnp.max(logits, axis=1, keepdims=True)
        e = jnp.exp(logits - m)
        lse = m + jnp.log(jnp.sum(e, axis=1, keepdims=True))
        out_ref[...] = logits - lse


def _final(a0v, a1v, hn2v, disv, b2v, batchv, Wc, bc):
    return pl.pallas_call(
        _final_body,
        grid=(_TC_GRID,),
        in_specs=[
            pl.BlockSpec((_BRV, 8 * _L), lambda i: (i, 0)),
            pl.BlockSpec((_BRV, 8 * _L), lambda i: (i, 0)),
            pl.BlockSpec((_BRV, 8 * _H), lambda i: (i, 0)),
            pl.BlockSpec((_BRV, 8 * _L), lambda i: (i, 0)),
            pl.BlockSpec((1, 8 * _H), lambda i: (0, 0)),
            pl.BlockSpec((_BRV, 8), lambda i: (i, 0)),
            pl.BlockSpec((_H, _CLS), lambda i: (0, 0)),
            pl.BlockSpec((1, _CLS), lambda i: (0, 0)),
        ],
        out_specs=pl.BlockSpec((_G, _CLS), lambda i: (0, 0)),
        out_shape=jax.ShapeDtypeStruct((_G, _CLS), jnp.float32),
        scratch_shapes=[
            pltpu.VMEM((_G, _H), jnp.float32),
            pltpu.VMEM((_G, _H), jnp.float32),
        ],
    )(a0v, a1v, hn2v, disv, b2v, batchv, Wc, bc)


def _view(p):
    return p.reshape(_N_VIEW, 8 * _L)


def _blockdiag8(W):
    k, m = W.shape
    mask = (jnp.arange(8 * k)[:, None] // k) == (jnp.arange(8 * m)[None, :] // m)
    return jnp.where(mask, jnp.tile(W, (8, 8)), 0.0)


def kernel(x, edge_index, batch, W1, b1, W2, b2, Wc, bc):
    ei = edge_index.astype(jnp.int32)
    d0, d1 = _deg_kernel(ei)

    xv = jnp.concatenate(
        [x, jnp.zeros((_N_ACC - _N, _D), jnp.float32)]).reshape(_N_VIEW, 8 * _D)
    batchv = jnp.concatenate(
        [batch.astype(jnp.int32), jnp.full((_N_ACC - _N,), _G, jnp.int32)]
    ).reshape(_N_VIEW, 8)
    b1v = jnp.tile(b1, 8).reshape(1, 8 * _H)
    b2v = jnp.tile(b2, 8).reshape(1, 8 * _H)
    W1v = _blockdiag8(W1)
    W2v = _blockdiag8(W2)
    hn1v, disv = _dense1(xv, W1v, _view(d0), _view(d1))
    a0, a1 = _edge_kernel(ei, hn1v.reshape(_N_ACC, _L))
    hn2v = _dense2(_view(a0), _view(a1), hn1v, disv, b1v, W2v)
    c0, c1 = _edge_kernel(ei, hn2v.reshape(_N_ACC, _L))
    return _final(_view(c0), _view(c1), hn2v, disv, b2v, batchv, Wc,
                  bc.reshape(1, _CLS))


# speedup vs baseline: 1.8399x; 1.0423x over previous
import functools

import jax
import jax.numpy as jnp
from jax import lax
from jax.experimental import pallas as pl
from jax.experimental.pallas import tpu as pltpu
from jax.experimental.pallas import tpu_sc as plsc

_N = 10000
_E = 320000
_D = 128
_H = 16
_G = 64
_CLS = 2

_NC = 2
_NS = 16
_L = 16

_N_ACC = 10112
_ACC_PER_SUB = _N_ACC // _NS
_E_CHUNK = 128
_N_CHUNKS = _E // _E_CHUNK
_CHUNK_BASE = _N_CHUNKS // (_NC * _NS)
_CHUNK_EXTRA = _N_CHUNKS - _CHUNK_BASE * _NC * _NS
_MAX_CHUNKS = _CHUNK_BASE + 1
_TRIPS = (_MAX_CHUNKS + 7) // 8

_N_VIEW = _N_ACC // 8
_BRV = 632
_BRL = _BRV * 8
_TC_GRID = _N_VIEW // _BRV

_NBUF = 8
_GLEAD = 4


def _sc_mesh():
    return plsc.VectorSubcoreMesh(core_axis_name="c", subcore_axis_name="s")


def _tile_chunks(gtile):
    n = jnp.where(gtile < _CHUNK_EXTRA, _CHUNK_BASE + 1, _CHUNK_BASE)
    base = gtile * _CHUNK_BASE + jnp.minimum(gtile, _CHUNK_EXTRA)
    return n, base


def _fill(ref, rows, value):
    def body(i, carry):
        ref[i] = jnp.full((_L,), value, jnp.float32)
        return carry

    lax.fori_loop(0, rows, body, 0)


def _acc_writeback(acc, out0, out1, c, s):
    sl = pl.ds(s * _ACC_PER_SUB, _ACC_PER_SUB)

    @pl.when(c == 0)
    def _w0():
        pltpu.sync_copy(acc.at[sl], out0.at[sl])

    @pl.when(c == 1)
    def _w1():
        pltpu.sync_copy(acc.at[sl], out1.at[sl])


def _stage_indices(ei_hbm, row, base, n_chunks, idx_v):
    @pl.when(n_chunks == _CHUNK_BASE)
    def _cp_base():
        pltpu.sync_copy(
            ei_hbm.at[row, pl.ds(base * _E_CHUNK, _CHUNK_BASE * _E_CHUNK)],
            idx_v.at[pl.ds(0, _CHUNK_BASE * _E_CHUNK)],
        )

    @pl.when(n_chunks == _MAX_CHUNKS)
    def _cp_extra():
        pltpu.sync_copy(
            ei_hbm.at[row, pl.ds(base * _E_CHUNK, _MAX_CHUNKS * _E_CHUNK)],
            idx_v.at[pl.ds(0, _MAX_CHUNKS * _E_CHUNK)],
        )


@functools.cache
def _make_deg_kernel():
    @functools.partial(
        pl.kernel,
        out_type=(
            jax.ShapeDtypeStruct((_N_ACC, _L), jnp.float32),
            jax.ShapeDtypeStruct((_N_ACC, _L), jnp.float32),
        ),
        mesh=_sc_mesh(),
        scratch_types=[
            pltpu.VMEM((_MAX_CHUNKS * _E_CHUNK,), jnp.int32),
            pltpu.VMEM((_E_CHUNK, _L), jnp.float32),
            pltpu.VMEM((_ACC_PER_SUB, _L), jnp.float32),
            pltpu.SemaphoreType.DMA,
            pltpu.VMEM_SHARED((_N_ACC, _L), jnp.float32),
        ],
        compiler_params=pltpu.CompilerParams(use_tc_tiling_on_sc=False),
    )
    def deg_kernel(ei_hbm, out0, out1, dst_v, ones_v, zeros_v, sem, acc):
        c = lax.axis_index("c")
        s = lax.axis_index("s")
        gtile = c * _NS + s
        n_chunks, base = _tile_chunks(gtile)
        _stage_indices(ei_hbm, 1, base, n_chunks, dst_v)
        _fill(ones_v, _E_CHUNK, 1.0)
        _fill(zeros_v, _ACC_PER_SUB, 0.0)
        pltpu.sync_copy(zeros_v, acc.at[pl.ds(s * _ACC_PER_SUB, _ACC_PER_SUB)])
        plsc.subcore_barrier()

        def body(j, carry):
            pltpu.async_copy(ones_v, acc.at[dst_v.at[pl.ds(j * _E_CHUNK, _E_CHUNK)]],
                             sem, add=True)
            return carry

        lax.fori_loop(0, n_chunks, body, 0)

        def drain(j, carry):
            pltpu.make_async_copy(ones_v, acc.at[dst_v.at[pl.ds(0, _E_CHUNK)]],
                                  sem).wait()
            return carry

        lax.fori_loop(0, n_chunks, drain, 0)
        plsc.subcore_barrier()
        _acc_writeback(acc, out0, out1, c, s)

    return deg_kernel


def _deg_kernel(edge_index):
    return _make_deg_kernel()(edge_index)


@functools.cache
def _make_edge_kernel():
    @functools.partial(
        pl.kernel,
        out_type=(
            jax.ShapeDtypeStruct((_N_ACC, _L), jnp.float32),
            jax.ShapeDtypeStruct((_N_ACC, _L), jnp.float32),
        ),
        mesh=_sc_mesh(),
        scratch_types=[
            pltpu.VMEM((_MAX_CHUNKS * _E_CHUNK,), jnp.int32),
            pltpu.VMEM((_MAX_CHUNKS * _E_CHUNK,), jnp.int32),
            pltpu.VMEM((_NBUF, _E_CHUNK, _L), jnp.float32),
            pltpu.VMEM((_ACC_PER_SUB, _L), jnp.float32),
        ]
        + [pltpu.SemaphoreType.DMA] * (2 * _NBUF)
        + [
            pltpu.VMEM_SHARED((_N_ACC, _L), jnp.float32),
            pltpu.VMEM_SHARED((_N_ACC, _L), jnp.float32),
        ],
        compiler_params=pltpu.CompilerParams(use_tc_tiling_on_sc=False),
    )
    def edge_kernel(ei_hbm, rows_hbm, out0, out1,
                    src_v, dst_v, rows_v, zeros_v,
                    g0, g1, g2, g3, g4, g5, g6, g7,
                    s0, s1, s2, s3, s4, s5, s6, s7,
                    acc, hn_s):
        gsems = (g0, g1, g2, g3, g4, g5, g6, g7)
        ssems = (s0, s1, s2, s3, s4, s5, s6, s7)
        c = lax.axis_index("c")
        s = lax.axis_index("s")
        gtile = c * _NS + s
        n_chunks, base = _tile_chunks(gtile)
        _stage_indices(ei_hbm, 0, base, n_chunks, src_v)
        _stage_indices(ei_hbm, 1, base, n_chunks, dst_v)
        _fill(zeros_v, _ACC_PER_SUB, 0.0)
        pltpu.sync_copy(zeros_v, acc.at[pl.ds(s * _ACC_PER_SUB, _ACC_PER_SUB)])
        stg = pl.ds(s * _ACC_PER_SUB, _ACC_PER_SUB)
        pltpu.sync_copy(rows_hbm.at[stg], hn_s.at[stg])
        plsc.subcore_barrier()

        def _sidx(j):
            return src_v.at[pl.ds(j * _E_CHUNK, _E_CHUNK)]

        def _didx(j):
            return dst_v.at[pl.ds(j * _E_CHUNK, _E_CHUNK)]

        for b in range(_GLEAD):
            pltpu.async_copy(hn_s.at[_sidx(b)], rows_v.at[b], gsems[b])

        def body(g, carry):
            for u in range(_NBUF):
                j = g * _NBUF + u

                @pl.when(j < n_chunks)
                def _slot():
                    pltpu.make_async_copy(hn_s.at[_sidx(j)], rows_v.at[u],
                                          gsems[u]).wait()
                    pltpu.async_copy(rows_v.at[u], acc.at[_didx(j)], ssems[u],
                                     add=True)
                    tgt = j + _GLEAD
                    tb = (u + _GLEAD) % _NBUF

                    @pl.when(tgt < n_chunks)
                    def _prefetch():
                        @pl.when(tgt >= _NBUF)
                        def _drain_prev():
                            pltpu.make_async_copy(rows_v.at[tb],
                                                  acc.at[_didx(0)],
                                                  ssems[tb]).wait()

                        pltpu.async_copy(hn_s.at[_sidx(tgt)], rows_v.at[tb],
                                         gsems[tb])

            return carry

        lax.fori_loop(0, _TRIPS, body, 0)
        for b in range(_NBUF):
            pltpu.make_async_copy(rows_v.at[b], acc.at[_didx(0)],
                                  ssems[b]).wait()
        plsc.subcore_barrier()
        _acc_writeback(acc, out0, out1, c, s)

    return edge_kernel


def _edge_kernel(edge_index, rows):
    return _make_edge_kernel()(edge_index, rows)


def _dense1_body(xv_ref, w1v_ref, d0_ref, d1_ref, hn_ref, dis_ref):
    hv = jnp.dot(xv_ref[...], w1v_ref[...], preferred_element_type=jnp.float32)
    deg = d0_ref[...] + d1_ref[...] + 1.0
    dis = lax.rsqrt(deg)
    dis_ref[...] = dis
    hn_ref[...] = hv * dis


def _dense1(xv, W1v, d0v, d1v):
    return pl.pallas_call(
        _dense1_body,
        grid=(_TC_GRID,),
        in_specs=[
            pl.BlockSpec((_BRV, 8 * _D), lambda i: (i, 0)),
            pl.BlockSpec((8 * _D, 8 * _H), lambda i: (0, 0)),
            pl.BlockSpec((_BRV, 8 * _L), lambda i: (i, 0)),
            pl.BlockSpec((_BRV, 8 * _L), lambda i: (i, 0)),
        ],
        out_specs=[
            pl.BlockSpec((_BRV, 8 * _H), lambda i: (i, 0)),
            pl.BlockSpec((_BRV, 8 * _L), lambda i: (i, 0)),
        ],
        out_shape=[
            jax.ShapeDtypeStruct((_N_VIEW, 8 * _H), jnp.float32),
            jax.ShapeDtypeStruct((_N_VIEW, 8 * _L), jnp.float32),
        ],
    )(xv, W1v, d0v, d1v)


def _dense2_body(a0_ref, a1_ref, hn1_ref, dis_ref, b1_ref, w2v_ref, hn2_ref):
    pre = (a0_ref[...] + a1_ref[...] + hn1_ref[...]) * dis_ref[...] + b1_ref[...]
    a = jnp.maximum(pre, 0.0)
    h2v = jnp.dot(a, w2v_ref[...], preferred_element_type=jnp.float32)
    hn2_ref[...] = h2v * dis_ref[...]


def _dense2(a0v, a1v, hn1v, disv, b1v, W2v):
    return pl.pallas_call(
        _dense2_body,
        grid=(_TC_GRID,),
        in_specs=[
            pl.BlockSpec((_BRV, 8 * _L), lambda i: (i, 0)),
            pl.BlockSpec((_BRV, 8 * _L), lambda i: (i, 0)),
            pl.BlockSpec((_BRV, 8 * _H), lambda i: (i, 0)),
            pl.BlockSpec((_BRV, 8 * _L), lambda i: (i, 0)),
            pl.BlockSpec((1, 8 * _H), lambda i: (0, 0)),
            pl.BlockSpec((8 * _H, 8 * _H), lambda i: (0, 0)),
        ],
        out_specs=pl.BlockSpec((_BRV, 8 * _H), lambda i: (i, 0)),
        out_shape=jax.ShapeDtypeStruct((_N_VIEW, 8 * _H), jnp.float32),
    )(a0v, a1v, hn1v, disv, b1v, W2v)


def _final_body(a0_ref, a1_ref, hn2_ref, dis_ref, b2_ref, batch_ref, wc_ref,
                bc_ref, out_ref, psum, pcnt):
    i = pl.program_id(0)

    @pl.when(i == 0)
    def _init():
        psum[...] = jnp.zeros_like(psum)
        pcnt[...] = jnp.zeros_like(pcnt)

    out2v = (a0_ref[...] + a1_ref[...] + hn2_ref[...]) * dis_ref[...] + b2_ref[...]
    b = batch_ref[...]
    iota_g = lax.broadcasted_iota(jnp.int32, (1, _G), 1)
    acc_s = psum[...]
    acc_c = pcnt[...]
    ones16 = jnp.ones((_BRV, _H), jnp.float32)
    for k in range(8):
        onehot = (b[:, k:k + 1] == iota_g).astype(jnp.float32)
        sl = out2v[:, k * _H:(k + 1) * _H]
        acc_s += lax.dot_general(onehot, sl, (((0,), (0,)), ((), ())),
                                 preferred_element_type=jnp.float32)
        acc_c += lax.dot_general(onehot, ones16, (((0,), (0,)), ((), ())),
                                 preferred_element_type=jnp.float32)
    psum[...] = acc_s
    pcnt[...] = acc_c

    @pl.when(i == _TC_GRID - 1)
    def _finish():
        pooled = psum[...] / jnp.maximum(pcnt[...], 1.0)
        logits = jnp.dot(pooled, wc_ref[...], preferred_element_type=jnp.float32)
        logits = logits + bc_ref[...]
        m = jnp.max(logits, axis=1, keepdims=True)
        e = jnp.exp(logits - m)
        lse = m + jnp.log(jnp.sum(e, axis=1, keepdims=True))
        out_ref[...] = logits - lse


def _final(a0v, a1v, hn2v, disv, b2v, batchv, Wc, bc):
    return pl.pallas_call(
        _final_body,
        grid=(_TC_GRID,),
        in_specs=[
            pl.BlockSpec((_BRV, 8 * _L), lambda i: (i, 0)),
            pl.BlockSpec((_BRV, 8 * _L), lambda i: (i, 0)),
            pl.BlockSpec((_BRV, 8 * _H), lambda i: (i, 0)),
            pl.BlockSpec((_BRV, 8 * _L), lambda i: (i, 0)),
            pl.BlockSpec((1, 8 * _H), lambda i: (0, 0)),
            pl.BlockSpec((_BRV, 8), lambda i: (i, 0)),
            pl.BlockSpec((_H, _CLS), lambda i: (0, 0)),
            pl.BlockSpec((1, _CLS), lambda i: (0, 0)),
        ],
        out_specs=pl.BlockSpec((_G, _CLS), lambda i: (0, 0)),
        out_shape=jax.ShapeDtypeStruct((_G, _CLS), jnp.float32),
        scratch_shapes=[
            pltpu.VMEM((_G, _H), jnp.float32),
            pltpu.VMEM((_G, _H), jnp.float32),
        ],
    )(a0v, a1v, hn2v, disv, b2v, batchv, Wc, bc)


def _view(p):
    return p.reshape(_N_VIEW, 8 * _L)


def _blockdiag8(W):
    k, m = W.shape
    mask = (jnp.arange(8 * k)[:, None] // k) == (jnp.arange(8 * m)[None, :] // m)
    return jnp.where(mask, jnp.tile(W, (8, 8)), 0.0)


def kernel(x, edge_index, batch, W1, b1, W2, b2, Wc, bc):
    ei = edge_index.astype(jnp.int32)
    d0, d1 = _deg_kernel(ei)

    xv = jnp.concatenate(
        [x, jnp.zeros((_N_ACC - _N, _D), jnp.float32)]).reshape(_N_VIEW, 8 * _D)
    batchv = jnp.concatenate(
        [batch.astype(jnp.int32), jnp.full((_N_ACC - _N,), _G, jnp.int32)]
    ).reshape(_N_VIEW, 8)
    b1v = jnp.tile(b1, 8).reshape(1, 8 * _H)
    b2v = jnp.tile(b2, 8).reshape(1, 8 * _H)
    W1v = _blockdiag8(W1)
    W2v = _blockdiag8(W2)
    hn1v, disv = _dense1(xv, W1v, _view(d0), _view(d1))
    a0, a1 = _edge_kernel(ei, hn1v.reshape(_N_ACC, _L))
    hn2v = _dense2(_view(a0), _view(a1), hn1v, disv, b1v, W2v)
    c0, c1 = _edge_kernel(ei, hn2v.reshape(_N_ACC, _L))
    return _final(_view(c0), _view(c1), hn2v, disv, b2v, batchv, Wc,
                  bc.reshape(1, _CLS))
